# Initial kernel scaffold; baseline (speedup 1.0000x reference)
import jax
import jax.numpy as jnp
from jax.experimental import pallas as pl


def kernel(dna, embedding_table):
    raise NotImplementedError("write your pallas kernel here")


# speedup vs baseline: 5.1394x; 5.1394x over previous
import functools

import jax
import jax.numpy as jnp
from jax import lax
from jax.experimental import pallas as pl
from jax.experimental.pallas import tpu as pltpu
from jax.experimental.pallas import tpu_sc as plsc

_NC, _NS, _L = 2, 16, 16
_NW = _NC * _NS
_TAB_PAD = 352


@functools.lru_cache(maxsize=None)
def _build(B, C):
    assert B % _NW == 0
    per_w = B // _NW
    assert per_w % C == 0 and C % _L == 0
    n_chunks = per_w // C
    mesh = plsc.VectorSubcoreMesh(core_axis_name="c", subcore_axis_name="s")

    @functools.partial(
        pl.kernel,
        out_type=jax.ShapeDtypeStruct((B * 4,), jnp.float32),
        mesh=mesh,
        scratch_types=[
            pltpu.VMEM((_TAB_PAD,), jnp.float32),
            pltpu.VMEM((C,), jnp.int32),
            pltpu.VMEM((C * 4,), jnp.float32),
        ],
        compiler_params=pltpu.CompilerParams(needs_layout_passes=False),
    )
    def k(ids_hbm, tab_hbm, out_hbm, tab_v, ids_v, out_v):
        wid = lax.axis_index("s") * _NC + lax.axis_index("c")
        base = wid * per_w
        pltpu.sync_copy(tab_hbm, tab_v)
        iota = lax.broadcasted_iota(jnp.int32, (_L,), 0)
        iota4 = iota * 4

        def chunk_body(j, carry):
            cb = base + j * C
            pltpu.sync_copy(ids_hbm.at[pl.ds(cb, C)], ids_v)

            def it_body(t, carry2):
                ids16 = ids_v[pl.ds(t * _L, _L)]
                ids4 = ids16 * 4
                for c in range(4):
                    vals = plsc.load_gather(tab_v, [ids4 + c])
                    plsc.store_scatter(out_v, [(t * (_L * 4) + c) + iota4], vals)
                return carry2

            lax.fori_loop(0, C // _L, it_body, 0)
            pltpu.sync_copy(out_v, out_hbm.at[pl.ds(cb * 4, C * 4)])
            return carry

        lax.fori_loop(0, n_chunks, chunk_body, 0)

    return k


def kernel(dna, embedding_table):
    nbatch, seqlen = dna.shape
    B = nbatch * seqlen
    ids = dna.reshape(B)
    tab = jnp.pad(embedding_table.reshape(-1),
                  (0, _TAB_PAD - embedding_table.size))
    out = _build(B, 10240)(ids, tab)
    return out.reshape(nbatch, seqlen, 1, 4)
